# Initial kernel scaffold; baseline (speedup 1.0000x reference)
import jax
import jax.numpy as jnp
from jax.experimental import pallas as pl


def kernel(x, table):
    raise NotImplementedError("write your pallas kernel here")


# speedup vs baseline: 4.5430x; 4.5430x over previous
import functools

import jax
import jax.numpy as jnp
from jax import lax
from jax.experimental import pallas as pl
from jax.experimental.pallas import tpu as pltpu
from jax.experimental.pallas import tpu_sc as plsc

_EMB = 64
_B = 4096 * 50

_NC = 2
_NS = 16
_NW = _NC * _NS
_BPW = _B // _NW
_C = 800
_NCHUNK = _BPW // _C


@functools.partial(
    pl.kernel,
    mesh=plsc.VectorSubcoreMesh(
        core_axis_name="c", subcore_axis_name="s", num_cores=_NC,
        num_subcores=_NS),
    out_type=jax.ShapeDtypeStruct((_B, _EMB), jnp.float32),
    scratch_types=[
        pltpu.VMEM((_C,), jnp.int32),
        pltpu.VMEM((_C, _EMB), jnp.float32),
        pltpu.SemaphoreType.DMA,
    ],
    compiler_params=pltpu.CompilerParams(use_tc_tiling_on_sc=False),
)
def _gather_kernel(table_hbm, idx_hbm, out_hbm, idx_v, rows_v, sem):
    wid = lax.axis_index("s") * _NC + lax.axis_index("c")
    base = wid * _BPW

    def body(g, carry):
        off = base + g * _C
        pltpu.sync_copy(idx_hbm.at[pl.ds(off, _C)], idx_v)
        pltpu.async_copy(table_hbm.at[idx_v], rows_v, sem).wait()
        pltpu.sync_copy(rows_v, out_hbm.at[pl.ds(off, _C)])
        return carry

    lax.fori_loop(0, _NCHUNK, body, 0)


def kernel(x, table):
    idx = x.reshape(-1).astype(jnp.int32)
    out = _gather_kernel(table, idx)
    return out.reshape(x.shape + (table.shape[1],))


# speedup vs baseline: 4.6252x; 1.0181x over previous
import functools

import jax
import jax.numpy as jnp
from jax import lax
from jax.experimental import pallas as pl
from jax.experimental.pallas import tpu as pltpu
from jax.experimental.pallas import tpu_sc as plsc

_EMB = 64
_B = 4096 * 50

_NC = 2
_NS = 16
_NW = _NC * _NS
_BPW = _B // _NW
_C = 800
_NCHUNK = _BPW // _C


@functools.partial(
    pl.kernel,
    mesh=plsc.VectorSubcoreMesh(
        core_axis_name="c", subcore_axis_name="s", num_cores=_NC,
        num_subcores=_NS),
    out_type=jax.ShapeDtypeStruct((_B, _EMB), jnp.float32),
    scratch_types=[
        pltpu.VMEM((_NCHUNK, _C), jnp.int32),
        pltpu.VMEM((2, _C, _EMB), jnp.float32),
        pltpu.SemaphoreType.DMA,
        pltpu.SemaphoreType.DMA,
        pltpu.SemaphoreType.DMA,
        pltpu.SemaphoreType.DMA,
    ],
    compiler_params=pltpu.CompilerParams(use_tc_tiling_on_sc=False),
)
def _gather_kernel(table_hbm, idx_hbm, out_hbm, idx_v, rows_v,
                   gsem0, gsem1, wsem0, wsem1):
    wid = lax.axis_index("s") * _NC + lax.axis_index("c")
    base = wid * _BPW

    pltpu.sync_copy(idx_hbm.at[wid], idx_v)

    gsems = (gsem0, gsem1)
    wsems = (wsem0, wsem1)
    gathers = [None, None]
    writes = [None, None]

    gathers[0] = pltpu.async_copy(
        table_hbm.at[idx_v.at[0]], rows_v.at[0], gsems[0])
    for g in range(_NCHUNK):
        b = g % 2
        nb = (g + 1) % 2
        if g + 1 < _NCHUNK:
            if writes[nb] is not None:
                writes[nb].wait()
            gathers[nb] = pltpu.async_copy(
                table_hbm.at[idx_v.at[g + 1]], rows_v.at[nb], gsems[nb])
        gathers[b].wait()
        writes[b] = pltpu.async_copy(
            rows_v.at[b], out_hbm.at[pl.ds(base + g * _C, _C)], wsems[b])
    writes[(_NCHUNK - 2) % 2].wait()
    writes[(_NCHUNK - 1) % 2].wait()


def kernel(x, table):
    idx = x.reshape(_NW, _NCHUNK, _C).astype(jnp.int32)
    out = _gather_kernel(table, idx)
    return out.reshape(x.shape + (table.shape[1],))
